# baseline (speedup 1.0000x reference)
import functools
import math

import jax
import jax.numpy as jnp
from jax import lax
from jax.experimental import pallas as pl
from jax.experimental.pallas import tpu as pltpu
from jax.experimental.pallas import tpu_sc as plsc

D_MODEL = 64
SCALE = math.sqrt(D_MODEL)

N_TOKENS = 4096 * 200
NUM_WORKERS = 32
PER_WORKER = N_TOKENS // NUM_WORKERS
CHUNK = 800
N_CHUNKS = PER_WORKER // CHUNK

_MESH = plsc.VectorSubcoreMesh(core_axis_name="c", subcore_axis_name="s")


@functools.partial(
    pl.kernel,
    out_type=jax.ShapeDtypeStruct((N_TOKENS, D_MODEL), jnp.float32),
    mesh=_MESH,
    scratch_types=[
        pltpu.VMEM((CHUNK,), jnp.int32),
        pltpu.VMEM((CHUNK, D_MODEL), jnp.float32),
        pltpu.SemaphoreType.DMA,
    ],
    compiler_params=pltpu.CompilerParams(use_tc_tiling_on_sc=False),
)
def _embed(x_hbm, w_hbm, out_hbm, idx_v, rows_v, sem):
    wid = lax.axis_index("s") * 2 + lax.axis_index("c")
    base = wid * PER_WORKER

    def chunk_body(g, carry):
        start = base + g * CHUNK
        pltpu.sync_copy(x_hbm.at[pl.ds(start, CHUNK)], idx_v)
        pltpu.async_copy(w_hbm.at[idx_v], rows_v, sem).wait()

        def scale_row(i, c):
            for j in range(D_MODEL // 16):
                sl = pl.ds(j * 16, 16)
                rows_v[i, sl] = rows_v[i, sl] * SCALE
            return c

        lax.fori_loop(0, CHUNK, scale_row, 0)
        pltpu.sync_copy(rows_v, out_hbm.at[pl.ds(start, CHUNK)])
        return carry

    lax.fori_loop(0, N_CHUNKS, chunk_body, 0)


def kernel(x, weight):
    out = _embed(x.reshape(-1), weight)
    return out.reshape(4096, 200, D_MODEL)


# speedup vs baseline: 1.1181x; 1.1181x over previous
import functools
import math

import jax
import jax.numpy as jnp
from jax import lax
from jax.experimental import pallas as pl
from jax.experimental.pallas import tpu as pltpu
from jax.experimental.pallas import tpu_sc as plsc

D_MODEL = 64
SCALE = math.sqrt(D_MODEL)

N_TOKENS = 4096 * 200
NUM_WORKERS = 32
PER_WORKER = N_TOKENS // NUM_WORKERS
CHUNK = 200
NBUF = 4
N_CHUNKS = PER_WORKER // CHUNK
N_ROUNDS = N_CHUNKS // NBUF

_MESH = plsc.VectorSubcoreMesh(core_axis_name="c", subcore_axis_name="s")


@functools.partial(
    pl.kernel,
    out_type=jax.ShapeDtypeStruct((N_TOKENS, D_MODEL), jnp.float32),
    mesh=_MESH,
    scratch_types=[
        pltpu.VMEM((PER_WORKER,), jnp.int32),
        [pltpu.VMEM((CHUNK, D_MODEL), jnp.float32) for _ in range(NBUF)],
        [pltpu.VMEM((CHUNK, D_MODEL), jnp.float32) for _ in range(NBUF)],
        [pltpu.SemaphoreType.DMA for _ in range(NBUF)],
        [pltpu.SemaphoreType.DMA for _ in range(NBUF)],
    ],
    compiler_params=pltpu.CompilerParams(use_tc_tiling_on_sc=False),
)
def _embed(x_hbm, w_hbm, out_hbm, idx_v, gbuf, sbuf, sem_g, sem_st):
    wid = lax.axis_index("s") * 2 + lax.axis_index("c")
    base = wid * PER_WORKER

    pltpu.sync_copy(x_hbm.at[pl.ds(base, PER_WORKER)], idx_v)

    def start_gather(g, b):
        idx = idx_v.at[pl.ds(g * CHUNK, CHUNK)]
        pltpu.async_copy(w_hbm.at[idx], gbuf[b], sem_g[b])

    def wait_gather(b):
        idx = idx_v.at[pl.ds(0, CHUNK)]
        pltpu.make_async_copy(w_hbm.at[idx], gbuf[b], sem_g[b]).wait()

    def start_store(g, b):
        pltpu.async_copy(
            sbuf[b], out_hbm.at[pl.ds(base + g * CHUNK, CHUNK)], sem_st[b]
        )

    def wait_store(b):
        pltpu.make_async_copy(
            sbuf[b], out_hbm.at[pl.ds(base, CHUNK)], sem_st[b]
        ).wait()

    def scale(b):
        src = gbuf[b]
        dst = sbuf[b]

        @plsc.parallel_loop(0, CHUNK, 1, unroll=8)
        def _(i):
            for j in range(D_MODEL // 16):
                sl = pl.ds(j * 16, 16)
                dst[i, sl] = src[i, sl] * SCALE

    for b in range(NBUF):
        start_gather(b, b)

    def round_body(gg, carry):
        for b in range(NBUF):
            g = gg * NBUF + b
            wait_gather(b)

            @pl.when(gg > 0)
            def _():
                wait_store(b)

            scale(b)

            @pl.when(gg + 1 < N_ROUNDS)
            def _():
                start_gather(g + NBUF, b)

            start_store(g, b)
        return carry

    lax.fori_loop(0, N_ROUNDS, round_body, 0)

    for b in range(NBUF):
        wait_store(b)


def kernel(x, weight):
    out = _embed(x.reshape(-1), weight)
    return out.reshape(4096, 200, D_MODEL)
